# baseline (speedup 1.0000x reference)
import functools

import jax
import jax.numpy as jnp
from jax import lax
from jax.experimental import pallas as pl
from jax.experimental.pallas import tpu as pltpu, tpu_sc as plsc

NC, NS, L = 2, 16, 16
NW = NC * NS
B = 16384
D = 16
BPW = B // NW
NCHUNK = 4
CHUNK = BPW // NCHUNK


def _sc_body(uidx_hbm, iidx_hbm, w_hbm, h_hbm, out_hbm,
             uidx_v, iidx_v, urows_v, vrows_v, pbuf_v, out_v, sem):
    wid = lax.axis_index("s") * NC + lax.axis_index("c")
    base = wid * BPW
    iota = lax.iota(jnp.int32, L)

    pltpu.sync_copy(uidx_hbm.at[pl.ds(wid * NCHUNK, NCHUNK), :], uidx_v)
    pltpu.sync_copy(iidx_hbm.at[pl.ds(wid * NCHUNK, NCHUNK), :], iidx_v)

    copies = []
    for j in range(NCHUNK):
        copies.append(pltpu.async_copy(w_hbm.at[uidx_v.at[j]], urows_v.at[j], sem))
        copies.append(pltpu.async_copy(h_hbm.at[iidx_v.at[j]], vrows_v.at[j], sem))
    for c in copies:
        c.wait()

    for j in range(NCHUNK):

        def body(i, _, j=j):
            for t in range(L):
                row = i * L + t
                p = urows_v[j, row, :] * vrows_v[j, row, :]
                pbuf_v[pl.ds(t * (D + 1), D)] = p
            acc = jnp.zeros((L,), jnp.float32)
            for k in range(D):
                acc = acc + plsc.load_gather(pbuf_v, [iota * (D + 1) + k])
            plsc.store_scatter(out_v, [j * CHUNK + i * L + iota], acc)
            return _

        lax.fori_loop(0, CHUNK // L, body, 0)

    pltpu.sync_copy(out_v, out_hbm.at[pl.ds(base, BPW)])


@jax.jit
def _mf_ips_sc(uidx, iidx, w, h):
    mesh = plsc.VectorSubcoreMesh(core_axis_name="c", subcore_axis_name="s")
    fn = pl.kernel(
        _sc_body,
        out_type=jax.ShapeDtypeStruct((B,), jnp.float32),
        mesh=mesh,
        compiler_params=pltpu.CompilerParams(
            needs_layout_passes=False,
            use_tc_tiling_on_sc=False,
        ),
        scratch_types=[
            pltpu.VMEM((NCHUNK, CHUNK), jnp.int32),
            pltpu.VMEM((NCHUNK, CHUNK), jnp.int32),
            pltpu.VMEM((NCHUNK, CHUNK, D), jnp.float32),
            pltpu.VMEM((NCHUNK, CHUNK, D), jnp.float32),
            pltpu.VMEM((L * (D + 1),), jnp.float32),
            pltpu.VMEM((BPW,), jnp.float32),
            pltpu.SemaphoreType.DMA,
        ],
    )
    return fn(uidx, iidx, w, h)


def kernel(x, W, H):
    uidx = x[:, 0].reshape(NW * NCHUNK, CHUNK)
    iidx = x[:, 1].reshape(NW * NCHUNK, CHUNK)
    return _mf_ips_sc(uidx, iidx, W, H)
